# baseline (speedup 1.0000x reference)
import jax
import jax.numpy as jnp
from jax import lax
from jax.experimental import pallas as pl

B_BLK = 512
NPAD = 128


def _tc_body(sid_ref, scale_ref, shift_ref, x_ref, out_ref):
    sid = sid_ref[0, 0, :]
    iota = lax.broadcasted_iota(jnp.int32, (1, NPAD), 1)
    onehot = (sid[:, None] == iota).astype(jnp.float32)
    scale_rows = jnp.dot(onehot, scale_ref[...], preferred_element_type=jnp.float32)
    shift_rows = jnp.dot(onehot, shift_ref[...], preferred_element_type=jnp.float32)
    out_ref[...] = scale_rows[:, None, :] * x_ref[...] + shift_rows[:, None, :]


def kernel(x, session_id, session_shift, session_scale):
    B, T, D = x.shape
    G = B // B_BLK
    sid3 = session_id.astype(jnp.int32).reshape(G, 1, B_BLK)
    n = session_scale.shape[0]
    scale_p = jnp.pad(session_scale, ((0, NPAD - n), (0, 0)))
    shift_p = jnp.pad(session_shift, ((0, NPAD - n), (0, 0)))
    return pl.pallas_call(
        _tc_body,
        grid=(G,),
        in_specs=[
            pl.BlockSpec((1, 1, B_BLK), lambda i: (i, 0, 0)),
            pl.BlockSpec((NPAD, D), lambda i: (0, 0)),
            pl.BlockSpec((NPAD, D), lambda i: (0, 0)),
            pl.BlockSpec((B_BLK, T, D), lambda i: (i, 0, 0)),
        ],
        out_specs=pl.BlockSpec((B_BLK, T, D), lambda i: (i, 0, 0)),
        out_shape=jax.ShapeDtypeStruct((B, T, D), jnp.float32),
    )(sid3, scale_p, shift_p, x)
